# baseline (speedup 1.0000x reference)
import functools

import jax
import jax.numpy as jnp
from jax import lax
from jax.experimental import pallas as pl
from jax.experimental.pallas import tpu as pltpu
from jax.experimental.pallas import tpu_sc as plsc

_B, _C, _N = 4, 768, 8192
_K = 8
_ROWS = _B * _C
_NW = 32
_RPW = _ROWS // _NW
_LANES = 16
_VPR = _N // _LANES
_OCTS = _VPR // 8
_FB_UNROLL = 4


def _vsort_desc(v):
    return plsc.sort_key_val(v, v, descending=True)[0]


def _combine(a, b, lane_lt8):
    return _vsort_desc(jnp.where(lane_lt8, a, lax.rev(b, (0,))))


def _insert(ts, x):
    out = []
    cur = x
    for t in ts:
        out.append(jnp.maximum(t, cur))
        cur = jnp.minimum(t, cur)
    return tuple(out)


def _merge_tree(vs, lane_lt8):
    vs = [_vsort_desc(t) for t in vs]
    while len(vs) > 1:
        vs = [_combine(vs[i], vs[i + 1], lane_lt8)
              for i in range(0, len(vs), 2)]
    return vs[0]


def _quad(a, b, c, d):
    m1, n1 = jnp.maximum(a, b), jnp.minimum(a, b)
    m2, n2 = jnp.maximum(c, d), jnp.minimum(c, d)
    w = jnp.maximum(m1, m2)
    sec = jnp.maximum(jnp.minimum(m1, m2), jnp.maximum(n1, n2))
    return w, sec


@functools.partial(
    pl.kernel,
    out_type=jax.ShapeDtypeStruct((_ROWS * _K,), jnp.float32),
    mesh=plsc.VectorSubcoreMesh(core_axis_name="c", subcore_axis_name="s"),
    scratch_types=[
        pltpu.VMEM((_N,), jnp.float32),
        pltpu.VMEM((_N,), jnp.float32),
        pltpu.VMEM((_N,), jnp.float32),
        pltpu.VMEM((_N,), jnp.float32),
        pltpu.VMEM((_N,), jnp.float32),
        pltpu.VMEM((_N,), jnp.float32),
        pltpu.VMEM((_LANES,), jnp.float32),
        pltpu.VMEM((_RPW * _K + _LANES - _K,), jnp.float32),
        pltpu.SemaphoreType.DMA,
        pltpu.SemaphoreType.DMA,
        pltpu.SemaphoreType.DMA,
        pltpu.SemaphoreType.DMA,
        pltpu.SemaphoreType.DMA,
        pltpu.SemaphoreType.DMA,
    ],
    compiler_params=pltpu.CompilerParams(needs_layout_passes=False),
)
def _topk_sc(x_hbm, out_hbm, buf0, buf1, buf2, buf3, buf4, buf5, s16, out_v, sem0, sem1, sem2, sem3, sem4, sem5):
    nc = 2
    wid = lax.axis_index("s") * nc + lax.axis_index("c")
    base = wid * _RPW
    lane = lax.iota(jnp.int32, 16)
    lane_lt8 = lane < _K
    seven = jnp.full((_LANES,), 7, jnp.int32)
    neg = jnp.full((_LANES,), -jnp.inf, jnp.float32)

    def row_compute(buf, row_local):
        def body(i, carry):
            ts, dmax = carry[:4], carry[4]
            for u in range(2):
                off = (2 * i + u) * 8 * _LANES
                v = [buf[pl.ds(off + j * _LANES, _LANES)] for j in range(8)]
                w1, s1 = _quad(*v[:4])
                w2, s2 = _quad(*v[4:])
                w = jnp.maximum(w1, w2)
                sec = jnp.maximum(jnp.minimum(w1, w2), jnp.maximum(s1, s2))
                dmax = jnp.maximum(dmax, sec)
                ts = _insert(ts, w)
            return ts + (dmax,)

        carry = lax.fori_loop(0, _OCTS // 2, body, (neg,) * 5)
        ts, dmax = carry[:4], carry[4]
        cand = _merge_tree(list(ts), lane_lt8)
        s16[...] = cand
        out8 = plsc.load_gather(s16, [seven])
        viol = jnp.any((ts[3] > out8) | (dmax > out8))

        def fallback():
            def fb_body(i, ts8):
                for j in range(_FB_UNROLL):
                    v = buf[pl.ds((i * _FB_UNROLL + j) * _LANES, _LANES)]
                    ts8 = _insert(ts8, v)
                return ts8
            ts8 = lax.fori_loop(0, _VPR // _FB_UNROLL, fb_body, (neg,) * _K)
            return _merge_tree(list(ts8), lane_lt8)

        final = lax.cond(viol, fallback, lambda: cand)
        plsc.store_compressed(out_v.at[pl.ds(row_local * _K, _LANES)],
                              final, mask=lane_lt8)

    bufs = (buf0, buf1, buf2, buf3, buf4, buf5)
    sems = (sem0, sem1, sem2, sem3, sem4, sem5)
    for b in range(6):
        pltpu.async_copy(x_hbm.at[base + b], bufs[b], sems[b])

    def step(st, carry):
        r0 = 6 * st
        for b in range(6):
            pltpu.make_async_copy(x_hbm.at[base + r0 + b], bufs[b],
                                  sems[b]).wait()
            row_compute(bufs[b], r0 + b)
            nxt = jnp.minimum(r0 + b + 6, _RPW - 1)
            pltpu.async_copy(x_hbm.at[base + nxt], bufs[b], sems[b])
        return carry

    lax.fori_loop(0, _RPW // 6, step, 0)

    for b in range(6):
        pltpu.make_async_copy(x_hbm.at[base], bufs[b], sems[b]).wait()

    pltpu.sync_copy(out_v.at[pl.ds(0, _RPW * _K)],
                    out_hbm.at[pl.ds(base * _K, _RPW * _K)])


def kernel(x):
    out = _topk_sc(x.reshape(_ROWS, _N))
    return out.reshape(_B, _C, _K)


# speedup vs baseline: 1.0131x; 1.0131x over previous
import functools

import jax
import jax.numpy as jnp
from jax import lax
from jax.experimental import pallas as pl
from jax.experimental.pallas import tpu as pltpu
from jax.experimental.pallas import tpu_sc as plsc

_B, _C, _N = 4, 768, 8192
_K = 8
_ROWS = _B * _C
_NW = 32
_RPW = _ROWS // _NW
_LANES = 16
_VPR = _N // _LANES
_OCTS = _VPR // 8
_FB_UNROLL = 4


def _vsort_desc(v):
    return plsc.sort_key_val(v, v, descending=True)[0]


def _combine(a, b, lane_lt8):
    return _vsort_desc(jnp.where(lane_lt8, a, lax.rev(b, (0,))))


def _insert(ts, x):
    out = []
    cur = x
    for t in ts:
        out.append(jnp.maximum(t, cur))
        cur = jnp.minimum(t, cur)
    return tuple(out)


def _merge_tree(vs, lane_lt8):
    vs = [_vsort_desc(t) for t in vs]
    while len(vs) > 1:
        vs = [_combine(vs[i], vs[i + 1], lane_lt8)
              for i in range(0, len(vs), 2)]
    return vs[0]


def _quad(a, b, c, d):
    m1, n1 = jnp.maximum(a, b), jnp.minimum(a, b)
    m2, n2 = jnp.maximum(c, d), jnp.minimum(c, d)
    w = jnp.maximum(m1, m2)
    sec = jnp.maximum(jnp.minimum(m1, m2), jnp.maximum(n1, n2))
    return w, sec


@functools.partial(
    pl.kernel,
    out_type=jax.ShapeDtypeStruct((_ROWS * _K,), jnp.float32),
    mesh=plsc.VectorSubcoreMesh(core_axis_name="c", subcore_axis_name="s"),
    scratch_types=[
        pltpu.VMEM((_N,), jnp.float32),
        pltpu.VMEM((_N,), jnp.float32),
        pltpu.VMEM((_N,), jnp.float32),
        pltpu.VMEM((_N,), jnp.float32),
        pltpu.VMEM((_LANES,), jnp.float32),
        pltpu.VMEM((_RPW * _K + _LANES - _K,), jnp.float32),
        pltpu.SemaphoreType.DMA,
        pltpu.SemaphoreType.DMA,
        pltpu.SemaphoreType.DMA,
        pltpu.SemaphoreType.DMA,
    ],
    compiler_params=pltpu.CompilerParams(needs_layout_passes=False),
)
def _topk_sc(x_hbm, out_hbm, buf0, buf1, buf2, buf3, s16, out_v, sem0, sem1, sem2, sem3):
    nc = 2
    wid = lax.axis_index("s") * nc + lax.axis_index("c")
    base = wid * _RPW
    lane = lax.iota(jnp.int32, 16)
    lane_lt8 = lane < _K
    seven = jnp.full((_LANES,), 7, jnp.int32)
    neg = jnp.full((_LANES,), -jnp.inf, jnp.float32)

    def row_compute(buf, row_local):
        def body(i, carry):
            ts, dmax = carry[:4], carry[4]
            for u in range(2):
                off = (2 * i + u) * 8 * _LANES
                v = [buf[pl.ds(off + j * _LANES, _LANES)] for j in range(8)]
                w1, s1 = _quad(*v[:4])
                w2, s2 = _quad(*v[4:])
                w = jnp.maximum(w1, w2)
                sec = jnp.maximum(jnp.minimum(w1, w2), jnp.maximum(s1, s2))
                dmax = jnp.maximum(dmax, sec)
                ts = _insert(ts, w)
            return ts + (dmax,)

        carry = lax.fori_loop(0, _OCTS // 2, body, (neg,) * 5)
        ts, dmax = carry[:4], carry[4]
        cand = _merge_tree(list(ts), lane_lt8)
        s16[...] = cand
        out8 = plsc.load_gather(s16, [seven])
        viol = jnp.any((ts[3] > out8) | (dmax > out8))

        def fallback():
            def fb_body(i, ts8):
                for j in range(_FB_UNROLL):
                    v = buf[pl.ds((i * _FB_UNROLL + j) * _LANES, _LANES)]
                    ts8 = _insert(ts8, v)
                return ts8
            ts8 = lax.fori_loop(0, _VPR // _FB_UNROLL, fb_body, (neg,) * _K)
            return _merge_tree(list(ts8), lane_lt8)

        final = lax.cond(viol, fallback, lambda: cand)
        plsc.store_compressed(out_v.at[pl.ds(row_local * _K, _LANES)],
                              final, mask=lane_lt8)

    bufs = (buf0, buf1, buf2, buf3)
    sems = (sem0, sem1, sem2, sem3)
    for b in range(4):
        pltpu.async_copy(x_hbm.at[base + b], bufs[b], sems[b])

    def step(st, carry):
        r0 = 4 * st
        for b in range(4):
            pltpu.make_async_copy(x_hbm.at[base + r0 + b], bufs[b],
                                  sems[b]).wait()
            row_compute(bufs[b], r0 + b)
            nxt = jnp.minimum(r0 + b + 4, _RPW - 1)
            pltpu.async_copy(x_hbm.at[base + nxt], bufs[b], sems[b])
        return carry

    lax.fori_loop(0, _RPW // 4, step, 0)

    for b in range(4):
        pltpu.make_async_copy(x_hbm.at[base], bufs[b], sems[b]).wait()

    pltpu.sync_copy(out_v.at[pl.ds(0, _RPW * _K)],
                    out_hbm.at[pl.ds(base * _K, _RPW * _K)])


def kernel(x):
    out = _topk_sc(x.reshape(_ROWS, _N))
    return out.reshape(_B, _C, _K)
